# baseline (speedup 1.0000x reference)
import functools

import jax
import jax.numpy as jnp
from jax import lax
from jax.experimental import pallas as pl
from jax.experimental.pallas import tpu as pltpu
from jax.experimental.pallas import tpu_sc as plsc

B = 16384
F = 26
FPAD = 32
INPUT_ROWS = 1000000
WPAD = 1000448
NC, NS, L = 2, 16, 16
NW = NC * NS
RPW = B // NW
IPW = RPW * F
CB = B // 128
CBW = RPW // 128

_mesh = plsc.VectorSubcoreMesh(core_axis_name="c", subcore_axis_name="s")


@functools.partial(
    pl.kernel,
    out_type=jax.ShapeDtypeStruct((B,), jnp.float32),
    mesh=_mesh,
    compiler_params=pltpu.CompilerParams(
        needs_layout_passes=False, use_tc_tiling_on_sc=False
    ),
    scratch_types=[
        pltpu.VMEM((FPAD // 8, CBW, 8, 128), jnp.int32),
        pltpu.VMEM((IPW,), jnp.float32),
        pltpu.VMEM((RPW,), jnp.float32),
        pltpu.VMEM((L,), jnp.float32),
        [pltpu.SemaphoreType.DMA] * CBW,
        pltpu.SemaphoreType.DMA,
    ],
)
def _lr_kernel(idx_hbm, w_hbm, b_hbm, out_hbm, idx_v, vals_v, out_v, b_v, sems, osem):
    wid = lax.axis_index("s") * NC + lax.axis_index("c")
    cb0 = wid * CBW
    JW = F * 128
    pltpu.sync_copy(idx_hbm.at[:, pl.ds(cb0, CBW), :, :], idx_v)
    pltpu.sync_copy(b_hbm, b_v)

    def fire(f, carry):
        rb = f // 8
        rr = lax.rem(f, 8)
        for j in range(CBW):
            iv = idx_v.at[rb, j, rr, :]
            pltpu.async_copy(
                w_hbm.at[iv], vals_v.at[pl.ds(j * JW + f * 128, 128)], sems[j]
            )
        return carry

    lax.fori_loop(0, F, fire, 0)

    bias = b_v[...]
    for j in range(CBW):
        pltpu.make_async_copy(
            w_hbm.at[pl.ds(0, JW)], vals_v.at[pl.ds(j * JW, JW)], sems[j]
        ).wait()

        def group(v, carry, j=j):
            acc = bias
            for f in range(F):
                acc = acc + vals_v[pl.ds(j * JW + f * 128 + v * L, L)]
            out_v[pl.ds(j * 128 + v * L, L)] = acc
            return carry

        lax.fori_loop(0, 128 // L, group, 0)
        pltpu.async_copy(
            out_v.at[pl.ds(j * 128, 128)],
            out_hbm.at[pl.ds(wid * RPW + j * 128, 128)],
            osem,
        )
    for j in range(CBW):
        pltpu.make_async_copy(
            out_v.at[pl.ds(j * 128, 128)],
            out_hbm.at[pl.ds(wid * RPW + j * 128, 128)],
            osem,
        ).wait()


def kernel(inputs, w, b):
    it = lax.pad(inputs.T, jnp.int32(0), ((0, FPAD - F, 0), (0, 0, 0)))
    idx = it.reshape(FPAD // 8, 8, CB, 128).transpose(0, 2, 1, 3)
    w_flat = lax.pad(w, jnp.float32(0), ((0, WPAD - INPUT_ROWS, 0), (0, 0, 0))).reshape(WPAD)
    b_vec = jnp.broadcast_to(b, (L,)).astype(jnp.float32)
    out = _lr_kernel(idx, w_flat, b_vec)
    return out.reshape(B, 1)


# speedup vs baseline: 1.2220x; 1.2220x over previous
import functools

import jax
import jax.numpy as jnp
from jax import lax
from jax.experimental import pallas as pl
from jax.experimental.pallas import tpu as pltpu
from jax.experimental.pallas import tpu_sc as plsc

B = 16384
F = 26
FPAD = 32
INPUT_ROWS = 1000000
WPAD = 1000448
NC, NS, L = 2, 16, 16
NW = NC * NS
RPW = B // NW
IPW = RPW * F
CB = B // 128
CBW = RPW // 128

_mesh = plsc.VectorSubcoreMesh(core_axis_name="c", subcore_axis_name="s")


@functools.partial(
    pl.kernel,
    out_type=jax.ShapeDtypeStruct((B,), jnp.float32),
    mesh=_mesh,
    compiler_params=pltpu.CompilerParams(
        needs_layout_passes=False, use_tc_tiling_on_sc=False
    ),
    scratch_types=[
        pltpu.VMEM((FPAD // 8, CBW, 8, 128), jnp.int32),
        pltpu.VMEM((IPW,), jnp.float32),
        pltpu.VMEM((RPW,), jnp.float32),
        pltpu.VMEM((L,), jnp.float32),
        pltpu.VMEM_SHARED((WPAD,), jnp.float32),
        pltpu.SemaphoreType.DMA,
        pltpu.SemaphoreType.DMA,
    ],
)
def _lr_kernel(idx_hbm, w_hbm, b_hbm, out_hbm, idx_v, vals_v, out_v, b_v, w_s, ssem, sem):
    sid = lax.axis_index("s")
    wid = sid * NC + lax.axis_index("c")
    cb0 = wid * CBW
    SPT = WPAD // NS
    pltpu.async_copy(
        w_hbm.at[pl.ds(sid * SPT, SPT)], w_s.at[pl.ds(sid * SPT, SPT)], ssem
    )
    pltpu.sync_copy(idx_hbm.at[:, pl.ds(cb0, CBW), :, :], idx_v)
    pltpu.sync_copy(b_hbm, b_v)
    pltpu.make_async_copy(
        w_hbm.at[pl.ds(sid * SPT, SPT)], w_s.at[pl.ds(sid * SPT, SPT)], ssem
    ).wait()
    plsc.subcore_barrier()

    def fire(f, carry):
        rb = f // 8
        rr = lax.rem(f, 8)
        for j in range(CBW):
            iv = idx_v.at[rb, j, rr, :]
            pltpu.async_copy(
                w_s.at[iv], vals_v.at[pl.ds(f * RPW + j * 128, 128)], sem
            )
        return carry

    lax.fori_loop(0, F, fire, 0)
    pltpu.make_async_copy(w_hbm.at[pl.ds(0, IPW)], vals_v, sem).wait()

    bias = b_v[...]

    def group(g, carry):
        acc = bias
        for f in range(F):
            acc = acc + vals_v[pl.ds(f * RPW + g * L, L)]
        out_v[pl.ds(g * L, L)] = acc
        return carry

    lax.fori_loop(0, RPW // L, group, 0)
    pltpu.sync_copy(out_v, out_hbm.at[pl.ds(wid * RPW, RPW)])


def kernel(inputs, w, b):
    it = lax.pad(inputs.T, jnp.int32(0), ((0, FPAD - F, 0), (0, 0, 0)))
    idx = it.reshape(FPAD // 8, 8, CB, 128).transpose(0, 2, 1, 3)
    w_flat = lax.pad(w, jnp.float32(0), ((0, WPAD - INPUT_ROWS, 0), (0, 0, 0))).reshape(WPAD)
    b_vec = jnp.broadcast_to(b, (L,)).astype(jnp.float32)
    out = _lr_kernel(idx, w_flat, b_vec)
    return out.reshape(B, 1)


# speedup vs baseline: 1.2241x; 1.0017x over previous
import functools

import jax
import jax.numpy as jnp
from jax import lax
from jax.experimental import pallas as pl
from jax.experimental.pallas import tpu as pltpu
from jax.experimental.pallas import tpu_sc as plsc

B = 16384
F = 26
FPAD = 32
INPUT_ROWS = 1000000
WPAD = 1000448
NC, NS, L = 2, 16, 16
NW = NC * NS
RPW = B // NW
IPW = RPW * F
CB = B // 128
CBW = RPW // 128

_mesh = plsc.VectorSubcoreMesh(core_axis_name="c", subcore_axis_name="s")


@functools.partial(
    pl.kernel,
    out_type=jax.ShapeDtypeStruct((B,), jnp.float32),
    mesh=_mesh,
    compiler_params=pltpu.CompilerParams(
        needs_layout_passes=False, use_tc_tiling_on_sc=False
    ),
    scratch_types=[
        pltpu.VMEM((FPAD // 8, CBW, 8, 128), jnp.int32),
        pltpu.VMEM((IPW,), jnp.float32),
        pltpu.VMEM((RPW,), jnp.float32),
        pltpu.VMEM((L,), jnp.float32),
        pltpu.VMEM_SHARED((WPAD,), jnp.float32),
        pltpu.SemaphoreType.DMA,
        pltpu.SemaphoreType.DMA,
    ],
)
def _lr_kernel(idx_hbm, w_hbm, b_hbm, out_hbm, idx_v, vals_v, out_v, b_v, w_s, ssem, sem):
    sid = lax.axis_index("s")
    wid = sid * NC + lax.axis_index("c")
    cb0 = wid * CBW
    SPT = WPAD // NS
    pltpu.async_copy(
        w_hbm.at[pl.ds(sid * SPT, SPT)], w_s.at[pl.ds(sid * SPT, SPT)], ssem
    )
    pltpu.sync_copy(idx_hbm.at[:, pl.ds(cb0, CBW), :, :], idx_v)
    pltpu.sync_copy(b_hbm, b_v)
    pltpu.make_async_copy(
        w_hbm.at[pl.ds(sid * SPT, SPT)], w_s.at[pl.ds(sid * SPT, SPT)], ssem
    ).wait()
    plsc.subcore_barrier()

    def fire(f, carry):
        rb = f // 8
        rr = lax.rem(f, 8)
        for j in range(CBW):
            iv = idx_v.at[rb, j, rr, :]
            pltpu.async_copy(
                w_s.at[iv], vals_v.at[pl.ds(f * RPW + j * 128, 128)], sem
            )
        return carry

    lax.fori_loop(0, F, fire, 0)
    pltpu.make_async_copy(w_hbm.at[pl.ds(0, IPW)], vals_v, sem).wait()

    bias = b_v[...]

    def group(g, carry):
        acc0 = bias + vals_v[pl.ds(0 * RPW + g * L, L)]
        acc1 = vals_v[pl.ds(1 * RPW + g * L, L)]
        for f in range(2, F, 2):
            acc0 = acc0 + vals_v[pl.ds(f * RPW + g * L, L)]
            acc1 = acc1 + vals_v[pl.ds((f + 1) * RPW + g * L, L)]
        out_v[pl.ds(g * L, L)] = acc0 + acc1
        return carry

    lax.fori_loop(0, RPW // L, group, 0)
    pltpu.sync_copy(out_v, out_hbm.at[pl.ds(wid * RPW, RPW)])


def kernel(inputs, w, b):
    it = lax.pad(inputs.T, jnp.int32(0), ((0, FPAD - F, 0), (0, 0, 0)))
    idx = it.reshape(FPAD // 8, 8, CB, 128).transpose(0, 2, 1, 3)
    w_flat = lax.pad(w, jnp.float32(0), ((0, WPAD - INPUT_ROWS, 0), (0, 0, 0))).reshape(WPAD)
    b_vec = jnp.broadcast_to(b, (L,)).astype(jnp.float32)
    out = _lr_kernel(idx, w_flat, b_vec)
    return out.reshape(B, 1)
